# baseline (speedup 1.0000x reference)
import math

import jax
import jax.numpy as jnp
from jax.experimental import pallas as pl
from jax.experimental.pallas import tpu as pltpu

K = 16


def _select_kernel(x0_ref, x1_ref, x2_ref, x3_ref, t_ref, out_ref, loss_ref):
    t = t_ref[0]
    refs = (x0_ref, x1_ref, x2_ref, x3_ref)
    partial = []
    for r in refs:
        d = r[0] - t[None]
        partial.append(jnp.sum(d * d, axis=(1, 2)))
    loss16 = jnp.concatenate(partial)

    n = t_ref.shape[1] * t_ref.shape[2]
    iota = jax.lax.broadcasted_iota(jnp.int32, (1, K), 1)[0]
    mn = jnp.min(loss16)
    idx = jnp.min(jnp.where(loss16 == mn, iota, K))

    loss_ref[0, 0, 0] = mn * (1.0 / n) + math.log(K, 2) / n

    q, j = idx // 4, idx % 4
    sel0 = x0_ref[0, pl.ds(j, 1)][0]
    sel1 = x1_ref[0, pl.ds(j, 1)][0]
    sel2 = x2_ref[0, pl.ds(j, 1)][0]
    sel3 = x3_ref[0, pl.ds(j, 1)][0]
    out_ref[0] = jnp.where(
        q == 0, sel0, jnp.where(q == 1, sel1, jnp.where(q == 2, sel2, sel3)))


def kernel(x, target):
    B, C, H, W = x.shape
    D = C // K
    N = D * H * W
    S = N // 128

    xr = x.reshape(B, K, S, 128)
    tr = target.reshape(B, S, 128)

    def xspec(q):
        return pl.BlockSpec((1, 4, S, 128), lambda b, q=q: (b, q, 0, 0))

    selected, min_loss = pl.pallas_call(
        _select_kernel,
        grid=(B,),
        in_specs=[xspec(0), xspec(1), xspec(2), xspec(3),
                  pl.BlockSpec((1, S, 128), lambda b: (b, 0, 0))],
        out_specs=[
            pl.BlockSpec((1, S, 128), lambda b: (b, 0, 0)),
            pl.BlockSpec((1, 1, 1), lambda b: (b, 0, 0),
                         memory_space=pltpu.SMEM),
        ],
        out_shape=[
            jax.ShapeDtypeStruct((B, S, 128), x.dtype),
            jax.ShapeDtypeStruct((B, 1, 1), x.dtype),
        ],
    )(xr, xr, xr, xr, tr)

    return selected.reshape(B, D, H, W), min_loss.reshape(B)
